# baseline (speedup 1.0000x reference)
import functools

import jax
import jax.numpy as jnp
from jax import lax
from jax.experimental import pallas as pl
from jax.experimental.pallas import tpu as pltpu
from jax.experimental.pallas import tpu_sc as plsc

N = 100000
E = 3200000
G = 128
F = 16

NC = 2
NS = 16
NW = NC * NS

NPAD = 100096
RPT = NPAD // NS
EPW = E // NW

CH = 2000
NIT = EPW // CH
CH2 = 400
NIT2 = EPW // CH2
SR2 = RPT // 16

f32 = jnp.float32

_SC_PARAMS = pltpu.CompilerParams(use_tc_tiling_on_sc=False)


def _mesh():
    return plsc.VectorSubcoreMesh(core_axis_name="c", subcore_axis_name="s")


@functools.partial(
    pl.kernel,
    out_type=jax.ShapeDtypeStruct((NC * NPAD,), f32),
    mesh=_mesh(),
    compiler_params=_SC_PARAMS,
    scratch_types=[
        pltpu.VMEM((CH,), jnp.int32),
        pltpu.VMEM((CH,), jnp.int32),
        pltpu.VMEM((CH,), f32),
        pltpu.VMEM((RPT,), f32),
        pltpu.VMEM_SHARED((NPAD,), f32),
        pltpu.SemaphoreType.DMA,
    ],
)
def _deg_sc(dste, zrow, ones_h, out, idx_a, idx_b, ones_v, stage_v, acc, sem_i):
    c = lax.axis_index("c")
    s = lax.axis_index("s")
    wid = s * NC + c
    r0 = s * RPT
    pltpu.sync_copy(zrow.at[pl.ds(r0, RPT)], stage_v)
    pltpu.sync_copy(stage_v, acc.at[pl.ds(r0, RPT)])
    pltpu.sync_copy(ones_h, ones_v)
    plsc.subcore_barrier()

    def start_idx(i, buf):
        base = wid * EPW + i * CH
        pltpu.async_copy(dste.at[pl.ds(base, CH)], buf, sem_i)

    def wait_idx(buf):
        pltpu.make_async_copy(dste.at[pl.ds(0, CH)], buf, sem_i).wait()

    start_idx(0, idx_a)
    start_idx(1, idx_b)

    def step(k, carry):
        i0 = 2 * k
        wait_idx(idx_a)
        pltpu.sync_copy(ones_v, acc.at[idx_a], add=True)
        start_idx(i0 + 2, idx_a)
        wait_idx(idx_b)
        pltpu.sync_copy(ones_v, acc.at[idx_b], add=True)
        start_idx(i0 + 3, idx_b)
        return carry

    lax.fori_loop(0, NIT // 2 - 1, step, 0)
    wait_idx(idx_a)
    pltpu.sync_copy(ones_v, acc.at[idx_a], add=True)
    wait_idx(idx_b)
    pltpu.sync_copy(ones_v, acc.at[idx_b], add=True)
    plsc.subcore_barrier()
    pltpu.sync_copy(acc.at[pl.ds(r0, RPT)], stage_v)
    pltpu.sync_copy(stage_v, out.at[pl.ds(c * NPAD + r0, RPT)])


@functools.partial(
    pl.kernel,
    out_type=jax.ShapeDtypeStruct((NC * NPAD, F), f32),
    mesh=_mesh(),
    compiler_params=_SC_PARAMS,
    scratch_types=[
        pltpu.VMEM((4, CH2), jnp.int32),
        pltpu.VMEM((4, CH2), jnp.int32),
        pltpu.VMEM((CH2, F), f32),
        pltpu.VMEM((CH2, F), f32),
        pltpu.VMEM((SR2, F), f32),
        pltpu.VMEM_SHARED((NPAD, F), f32),
        pltpu.SemaphoreType.DMA,
        pltpu.SemaphoreType.DMA,
        pltpu.SemaphoreType.DMA,
    ],
)
def _mp_sc(srce, dste, table, z16, out, src_r, dst_r,
           rows_a, rows_b, stage_v, acc, sem_i, sem_g, sem_s):
    c = lax.axis_index("c")
    s = lax.axis_index("s")
    wid = s * NC + c
    r0 = s * RPT

    def zstep(j, carry):
        h0 = r0 + j * SR2
        pltpu.sync_copy(z16.at[pl.ds(h0, SR2)], stage_v)
        pltpu.sync_copy(stage_v, acc.at[pl.ds(h0, SR2)])
        return carry

    lax.fori_loop(0, 16, zstep, 0)
    plsc.subcore_barrier()

    rows = (rows_a, rows_b)

    def start_idx(i, b4):
        base = wid * EPW + i * CH2
        pltpu.async_copy(srce.at[pl.ds(base, CH2)], src_r.at[b4], sem_i)
        pltpu.async_copy(dste.at[pl.ds(base, CH2)], dst_r.at[b4], sem_i)

    def wait_idx(b4):
        pltpu.make_async_copy(srce.at[pl.ds(0, CH2)], src_r.at[b4], sem_i).wait()
        pltpu.make_async_copy(dste.at[pl.ds(0, CH2)], dst_r.at[b4], sem_i).wait()

    def start_gather(b4, b2):
        pltpu.async_copy(table.at[src_r.at[b4]], rows[b2], sem_g)

    def wait_gather(b4, b2):
        pltpu.make_async_copy(table.at[src_r.at[b4]], rows[b2], sem_g).wait()

    def start_scatter(b4, b2):
        pltpu.async_copy(rows[b2], acc.at[dst_r.at[b4]], sem_s, add=True)

    def wait_scatter(b4, b2):
        pltpu.make_async_copy(rows[b2], acc.at[dst_r.at[b4]], sem_s).wait()

    start_idx(0, 0)
    start_idx(1, 1)
    start_idx(2, 2)
    wait_idx(0)
    start_gather(0, 0)

    def step(k, carry):
        for off in range(4):
            i = 4 * k + off
            b2 = off % 2
            b4 = off
            nb2 = (off + 1) % 2
            nb4 = (off + 1) % 4
            pb4 = (off + 3) % 4

            @pl.when(i < NIT2)
            def _():
                wait_gather(b4, b2)
                start_scatter(b4, b2)

                @pl.when(i >= 1)
                def _():
                    wait_scatter(pb4, nb2)

                @pl.when(i + 1 < NIT2)
                def _():
                    wait_idx(nb4)
                    start_gather(nb4, nb2)

                @pl.when(i + 3 < NIT2)
                def _():
                    start_idx(i + 3, pb4)

        return carry

    lax.fori_loop(0, (NIT2 + 3) // 4, step, 0)
    wait_scatter((NIT2 - 1) % 4, (NIT2 - 1) % 2)
    plsc.subcore_barrier()

    def ostep(j, carry):
        h0 = r0 + j * SR2
        pltpu.sync_copy(acc.at[pl.ds(h0, SR2)], stage_v)
        pltpu.sync_copy(stage_v, out.at[pl.ds(c * NPAD + h0, SR2)])
        return carry

    lax.fori_loop(0, 16, ostep, 0)


BT = 3128
NBT = NPAD // BT


def _prep_tc_body(i0_ref, i1_ref, x4_ref, dinv_ref, xs_ref):
    drow = lax.rsqrt(i0_ref[0] + i1_ref[0] + 1.0)
    dinv = jnp.transpose(drow, (1, 0))
    dinv_ref[...] = dinv
    x4s = x4_ref[...] * dinv
    xs_ref[...] = jnp.pad(x4s, ((0, 0), (0, F - 4)))


_prep_tc = pl.pallas_call(
    _prep_tc_body,
    grid=(NBT,),
    in_specs=[
        pl.BlockSpec((1, 1, BT), lambda i: (i, 0, 0)),
        pl.BlockSpec((1, 1, BT), lambda i: (NBT + i, 0, 0)),
        pl.BlockSpec((BT, 4), lambda i: (i, 0)),
    ],
    out_specs=[
        pl.BlockSpec((BT, 1), lambda i: (i, 0)),
        pl.BlockSpec((BT, F), lambda i: (i, 0)),
    ],
    out_shape=[
        jax.ShapeDtypeStruct((NPAD, 1), f32),
        jax.ShapeDtypeStruct((NPAD, F), f32),
    ],
)


def _mid_tc_body(a0_ref, a1_ref, x4_ref, dinv_ref, w1_ref, b1_ref, p_ref):
    dinv = dinv_ref[...]
    a = a0_ref[...] + a1_ref[...]
    macc = dinv * a[:, :3] + (dinv * dinv) * x4_ref[:, :3]
    h = jnp.dot(macc, w1_ref[...], preferred_element_type=f32,
                precision=lax.Precision.HIGHEST) + b1_ref[...]
    p_ref[...] = dinv * jnp.maximum(h, 0.0)


_mid_tc = pl.pallas_call(
    _mid_tc_body,
    grid=(NBT,),
    in_specs=[
        pl.BlockSpec((BT, F), lambda i: (i, 0)),
        pl.BlockSpec((BT, F), lambda i: (NBT + i, 0)),
        pl.BlockSpec((BT, 4), lambda i: (i, 0)),
        pl.BlockSpec((BT, 1), lambda i: (i, 0)),
        pl.BlockSpec((3, 16), lambda i: (0, 0)),
        pl.BlockSpec((1, 16), lambda i: (0, 0)),
    ],
    out_specs=pl.BlockSpec((BT, F), lambda i: (i, 0)),
    out_shape=jax.ShapeDtypeStruct((NPAD, F), f32),
)


def _fin_tc_body(a0_ref, a1_ref, p_ref, dinv_ref, brow_ref, w2_ref, b2_ref,
                 wl_ref, bl_ref, out_ref, gacc, cacc):
    i = pl.program_id(0)

    @pl.when(i == 0)
    def _():
        gacc[...] = jnp.zeros_like(gacc)
        cacc[...] = jnp.zeros_like(cacc)

    t = dinv_ref[...] * (a0_ref[...] + a1_ref[...] + p_ref[...])
    kk = lax.broadcasted_iota(jnp.int32, (G, BT), 0)
    oh = jnp.where(kk == brow_ref[0], 1.0, 0.0)
    gacc[...] += jnp.dot(oh, t, preferred_element_type=f32,
                         precision=lax.Precision.HIGHEST)
    cacc[...] += jnp.sum(oh, axis=1, keepdims=True)

    @pl.when(i == NBT - 1)
    def _():
        g = gacc[...]
        cnt = cacc[...]
        logits = jnp.dot(
            jnp.dot(g, w2_ref[...], preferred_element_type=f32,
                    precision=lax.Precision.HIGHEST)
            + cnt * b2_ref[...],
            wl_ref[...], preferred_element_type=f32,
            precision=lax.Precision.HIGHEST) + bl_ref[...]
        m = jnp.max(logits, axis=1, keepdims=True)
        z = logits - m
        out_ref[...] = z - jnp.log(jnp.sum(jnp.exp(z), axis=1, keepdims=True))


_fin_tc = pl.pallas_call(
    _fin_tc_body,
    grid=(NBT,),
    in_specs=[
        pl.BlockSpec((BT, F), lambda i: (i, 0)),
        pl.BlockSpec((BT, F), lambda i: (NBT + i, 0)),
        pl.BlockSpec((BT, F), lambda i: (i, 0)),
        pl.BlockSpec((BT, 1), lambda i: (i, 0)),
        pl.BlockSpec((1, 1, BT), lambda i: (i, 0, 0)),
        pl.BlockSpec((16, 16), lambda i: (0, 0)),
        pl.BlockSpec((1, 16), lambda i: (0, 0)),
        pl.BlockSpec((16, 7), lambda i: (0, 0)),
        pl.BlockSpec((1, 7), lambda i: (0, 0)),
    ],
    out_specs=pl.BlockSpec((G, 7), lambda i: (0, 0)),
    out_shape=jax.ShapeDtypeStruct((G, 7), f32),
    scratch_shapes=[
        pltpu.VMEM((G, F), f32),
        pltpu.VMEM((G, 1), f32),
    ],
)


def kernel(x, edge_index, edge_attr, batch, W1, b1, W2, b2, Wlin, blin):
    del edge_attr
    srce = edge_index[0]
    dste = edge_index[1]

    zrow = jnp.zeros((NPAD,), f32)
    ones_h = jnp.ones((CH,), f32)
    deg_parts = _deg_sc(dste, zrow, ones_h)
    deg2 = deg_parts.reshape(NC * NBT, 1, BT)

    x4 = jnp.zeros((NPAD, 4), f32).at[:N, :3].set(x)
    dinv, xs = _prep_tc(deg2, deg2, x4)

    z16 = jnp.zeros((NPAD, F), f32)
    acc1 = _mp_sc(srce, dste, xs, z16)

    p = _mid_tc(acc1, acc1, x4, dinv, W1, b1.reshape(1, 16))

    acc2 = _mp_sc(srce, dste, p, z16)

    brow = jnp.full((NPAD,), G, jnp.int32).at[:N].set(batch)
    brow = brow.reshape(NBT, 1, BT)
    return _fin_tc(acc2, acc2, p, dinv, brow, W2, b2.reshape(1, 16),
                   Wlin, blin.reshape(1, 7))


# speedup vs baseline: 1.0054x; 1.0054x over previous
import functools

import jax
import jax.numpy as jnp
from jax import lax
from jax.experimental import pallas as pl
from jax.experimental.pallas import tpu as pltpu
from jax.experimental.pallas import tpu_sc as plsc

N = 100000
E = 3200000
G = 128
F = 16

NC = 2
NS = 16
NW = NC * NS

NPAD = 100096
RPT = NPAD // NS
EPW = E // NW

CH = 2000
NIT = EPW // CH
CH2 = 400
NIT2 = EPW // CH2
SR2 = RPT // 16

f32 = jnp.float32

_SC_PARAMS = pltpu.CompilerParams(use_tc_tiling_on_sc=False)


def _mesh():
    return plsc.VectorSubcoreMesh(core_axis_name="c", subcore_axis_name="s")


@functools.partial(
    pl.kernel,
    out_type=jax.ShapeDtypeStruct((NC * NPAD,), f32),
    mesh=_mesh(),
    compiler_params=_SC_PARAMS,
    scratch_types=[
        pltpu.VMEM((CH,), jnp.int32),
        pltpu.VMEM((CH,), jnp.int32),
        pltpu.VMEM((CH,), f32),
        pltpu.VMEM((RPT,), f32),
        pltpu.VMEM_SHARED((NPAD,), f32),
        pltpu.SemaphoreType.DMA,
    ],
)
def _deg_sc(ei, zrow, ones_h, out, idx_a, idx_b, ones_v, stage_v, acc, sem_i):
    c = lax.axis_index("c")
    s = lax.axis_index("s")
    wid = s * NC + c
    r0 = s * RPT
    pltpu.sync_copy(zrow.at[pl.ds(r0, RPT)], stage_v)
    pltpu.sync_copy(stage_v, acc.at[pl.ds(r0, RPT)])
    pltpu.sync_copy(ones_h, ones_v)
    plsc.subcore_barrier()

    def start_idx(i, buf):
        base = wid * EPW + i * CH
        pltpu.async_copy(ei.at[pl.ds(E + base, CH)], buf, sem_i)

    def wait_idx(buf):
        pltpu.make_async_copy(ei.at[pl.ds(0, CH)], buf, sem_i).wait()

    start_idx(0, idx_a)
    start_idx(1, idx_b)

    def step(k, carry):
        i0 = 2 * k
        wait_idx(idx_a)
        pltpu.sync_copy(ones_v, acc.at[idx_a], add=True)
        start_idx(i0 + 2, idx_a)
        wait_idx(idx_b)
        pltpu.sync_copy(ones_v, acc.at[idx_b], add=True)
        start_idx(i0 + 3, idx_b)
        return carry

    lax.fori_loop(0, NIT // 2 - 1, step, 0)
    wait_idx(idx_a)
    pltpu.sync_copy(ones_v, acc.at[idx_a], add=True)
    wait_idx(idx_b)
    pltpu.sync_copy(ones_v, acc.at[idx_b], add=True)
    plsc.subcore_barrier()
    pltpu.sync_copy(acc.at[pl.ds(r0, RPT)], stage_v)
    pltpu.sync_copy(stage_v, out.at[pl.ds(c * NPAD + r0, RPT)])


@functools.partial(
    pl.kernel,
    out_type=jax.ShapeDtypeStruct((NC * NPAD, F), f32),
    mesh=_mesh(),
    compiler_params=_SC_PARAMS,
    scratch_types=[
        pltpu.VMEM((4, CH2), jnp.int32),
        pltpu.VMEM((4, CH2), jnp.int32),
        pltpu.VMEM((CH2, F), f32),
        pltpu.VMEM((CH2, F), f32),
        pltpu.VMEM((SR2, F), f32),
        pltpu.VMEM_SHARED((NPAD, F), f32),
        pltpu.SemaphoreType.DMA,
        pltpu.SemaphoreType.DMA,
        pltpu.SemaphoreType.DMA,
    ],
)
def _mp_sc(ei, table, z16, out, src_r, dst_r,
           rows_a, rows_b, stage_v, acc, sem_i, sem_g, sem_s):
    c = lax.axis_index("c")
    s = lax.axis_index("s")
    wid = s * NC + c
    r0 = s * RPT

    def zstep(j, carry):
        h0 = r0 + j * SR2
        pltpu.sync_copy(z16.at[pl.ds(h0, SR2)], stage_v)
        pltpu.sync_copy(stage_v, acc.at[pl.ds(h0, SR2)])
        return carry

    lax.fori_loop(0, 16, zstep, 0)
    plsc.subcore_barrier()

    rows = (rows_a, rows_b)

    def start_idx(i, b4):
        base = wid * EPW + i * CH2
        pltpu.async_copy(ei.at[pl.ds(base, CH2)], src_r.at[b4], sem_i)
        pltpu.async_copy(ei.at[pl.ds(E + base, CH2)], dst_r.at[b4], sem_i)

    def wait_idx(b4):
        pltpu.make_async_copy(ei.at[pl.ds(0, CH2)], src_r.at[b4], sem_i).wait()
        pltpu.make_async_copy(ei.at[pl.ds(0, CH2)], dst_r.at[b4], sem_i).wait()

    def start_gather(b4, b2):
        pltpu.async_copy(table.at[src_r.at[b4]], rows[b2], sem_g)

    def wait_gather(b4, b2):
        pltpu.make_async_copy(table.at[src_r.at[b4]], rows[b2], sem_g).wait()

    def start_scatter(b4, b2):
        pltpu.async_copy(rows[b2], acc.at[dst_r.at[b4]], sem_s, add=True)

    def wait_scatter(b4, b2):
        pltpu.make_async_copy(rows[b2], acc.at[dst_r.at[b4]], sem_s).wait()

    start_idx(0, 0)
    start_idx(1, 1)
    start_idx(2, 2)
    wait_idx(0)
    start_gather(0, 0)

    def step(k, carry):
        for off in range(4):
            i = 4 * k + off
            b2 = off % 2
            b4 = off
            nb2 = (off + 1) % 2
            nb4 = (off + 1) % 4
            pb4 = (off + 3) % 4

            @pl.when(i < NIT2)
            def _():
                wait_gather(b4, b2)
                start_scatter(b4, b2)

                @pl.when(i >= 1)
                def _():
                    wait_scatter(pb4, nb2)

                @pl.when(i + 1 < NIT2)
                def _():
                    wait_idx(nb4)
                    start_gather(nb4, nb2)

                @pl.when(i + 3 < NIT2)
                def _():
                    start_idx(i + 3, pb4)

        return carry

    lax.fori_loop(0, (NIT2 + 3) // 4, step, 0)
    wait_scatter((NIT2 - 1) % 4, (NIT2 - 1) % 2)
    plsc.subcore_barrier()

    def ostep(j, carry):
        h0 = r0 + j * SR2
        pltpu.sync_copy(acc.at[pl.ds(h0, SR2)], stage_v)
        pltpu.sync_copy(stage_v, out.at[pl.ds(c * NPAD + h0, SR2)])
        return carry

    lax.fori_loop(0, 16, ostep, 0)


BT = 3128
NBT = NPAD // BT


def _prep_tc_body(i0_ref, i1_ref, x4_ref, dinv_ref, xs_ref):
    drow = lax.rsqrt(i0_ref[0] + i1_ref[0] + 1.0)
    dinv = jnp.transpose(drow, (1, 0))
    dinv_ref[...] = dinv
    x4s = x4_ref[...] * dinv
    xs_ref[...] = jnp.pad(x4s, ((0, 0), (0, F - 4)))


_prep_tc = pl.pallas_call(
    _prep_tc_body,
    grid=(NBT,),
    in_specs=[
        pl.BlockSpec((1, 1, BT), lambda i: (i, 0, 0)),
        pl.BlockSpec((1, 1, BT), lambda i: (NBT + i, 0, 0)),
        pl.BlockSpec((BT, 4), lambda i: (i, 0)),
    ],
    out_specs=[
        pl.BlockSpec((BT, 1), lambda i: (i, 0)),
        pl.BlockSpec((BT, F), lambda i: (i, 0)),
    ],
    out_shape=[
        jax.ShapeDtypeStruct((NPAD, 1), f32),
        jax.ShapeDtypeStruct((NPAD, F), f32),
    ],
)


def _mid_tc_body(a0_ref, a1_ref, x4_ref, dinv_ref, w1_ref, b1_ref, p_ref):
    dinv = dinv_ref[...]
    a = a0_ref[...] + a1_ref[...]
    macc = dinv * a[:, :3] + (dinv * dinv) * x4_ref[:, :3]
    h = jnp.dot(macc, w1_ref[...], preferred_element_type=f32,
                precision=lax.Precision.HIGHEST) + b1_ref[...]
    p_ref[...] = dinv * jnp.maximum(h, 0.0)


_mid_tc = pl.pallas_call(
    _mid_tc_body,
    grid=(NBT,),
    in_specs=[
        pl.BlockSpec((BT, F), lambda i: (i, 0)),
        pl.BlockSpec((BT, F), lambda i: (NBT + i, 0)),
        pl.BlockSpec((BT, 4), lambda i: (i, 0)),
        pl.BlockSpec((BT, 1), lambda i: (i, 0)),
        pl.BlockSpec((3, 16), lambda i: (0, 0)),
        pl.BlockSpec((1, 16), lambda i: (0, 0)),
    ],
    out_specs=pl.BlockSpec((BT, F), lambda i: (i, 0)),
    out_shape=jax.ShapeDtypeStruct((NPAD, F), f32),
)


def _fin_tc_body(a0_ref, a1_ref, p_ref, dinv_ref, brow_ref, w2_ref, b2_ref,
                 wl_ref, bl_ref, out_ref, gacc, cacc):
    i = pl.program_id(0)

    @pl.when(i == 0)
    def _():
        gacc[...] = jnp.zeros_like(gacc)
        cacc[...] = jnp.zeros_like(cacc)

    t = dinv_ref[...] * (a0_ref[...] + a1_ref[...] + p_ref[...])
    kk = lax.broadcasted_iota(jnp.int32, (G, BT), 0)
    oh = jnp.where(kk == brow_ref[0], 1.0, 0.0)
    gacc[...] += jnp.dot(oh, t, preferred_element_type=f32,
                         precision=lax.Precision.HIGHEST)
    cacc[...] += jnp.sum(oh, axis=1, keepdims=True)

    @pl.when(i == NBT - 1)
    def _():
        g = gacc[...]
        cnt = cacc[...]
        logits = jnp.dot(
            jnp.dot(g, w2_ref[...], preferred_element_type=f32,
                    precision=lax.Precision.HIGHEST)
            + cnt * b2_ref[...],
            wl_ref[...], preferred_element_type=f32,
            precision=lax.Precision.HIGHEST) + bl_ref[...]
        m = jnp.max(logits, axis=1, keepdims=True)
        z = logits - m
        out_ref[...] = z - jnp.log(jnp.sum(jnp.exp(z), axis=1, keepdims=True))


_fin_tc = pl.pallas_call(
    _fin_tc_body,
    grid=(NBT,),
    in_specs=[
        pl.BlockSpec((BT, F), lambda i: (i, 0)),
        pl.BlockSpec((BT, F), lambda i: (NBT + i, 0)),
        pl.BlockSpec((BT, F), lambda i: (i, 0)),
        pl.BlockSpec((BT, 1), lambda i: (i, 0)),
        pl.BlockSpec((1, 1, BT), lambda i: (i, 0, 0)),
        pl.BlockSpec((16, 16), lambda i: (0, 0)),
        pl.BlockSpec((1, 16), lambda i: (0, 0)),
        pl.BlockSpec((16, 7), lambda i: (0, 0)),
        pl.BlockSpec((1, 7), lambda i: (0, 0)),
    ],
    out_specs=pl.BlockSpec((G, 7), lambda i: (0, 0)),
    out_shape=jax.ShapeDtypeStruct((G, 7), f32),
    scratch_shapes=[
        pltpu.VMEM((G, F), f32),
        pltpu.VMEM((G, 1), f32),
    ],
)


def kernel(x, edge_index, edge_attr, batch, W1, b1, W2, b2, Wlin, blin):
    del edge_attr
    ei = edge_index.reshape(2 * E)

    zrow = jnp.zeros((NPAD,), f32)
    ones_h = jnp.ones((CH,), f32)
    deg_parts = _deg_sc(ei, zrow, ones_h)
    deg2 = deg_parts.reshape(NC * NBT, 1, BT)

    x4 = jnp.zeros((NPAD, 4), f32).at[:N, :3].set(x)
    dinv, xs = _prep_tc(deg2, deg2, x4)

    z16 = jnp.zeros((NPAD, F), f32)
    acc1 = _mp_sc(ei, xs, z16)

    p = _mid_tc(acc1, acc1, x4, dinv, W1, b1.reshape(1, 16))

    acc2 = _mp_sc(ei, p, z16)

    brow = jnp.full((NPAD,), G, jnp.int32).at[:N].set(batch)
    brow = brow.reshape(NBT, 1, BT)
    return _fin_tc(acc2, acc2, p, dinv, brow, W2, b2.reshape(1, 16),
                   Wlin, blin.reshape(1, 7))
